# baseline (speedup 1.0000x reference)
import jax
import jax.numpy as jnp
from jax import lax
from jax.experimental import pallas as pl
from jax.experimental.pallas import tpu as pltpu
from jax.experimental.pallas import tpu_sc as plsc

_N = 100000
_NB = 256
_BS = 1.0 / _NB
_INV_BS = float(_NB)
_MIN_SIZE = _BS * 1.4142135
_SCALE = 1.0 / (_BS * _BS * 100.0)
_NW = 32
_CHUNK = 3128
_LAST = _N - (_NW - 1) * _CHUNK
_GROUPS = _CHUNK // 16
_TAIL = _CHUNK - _GROUPS * 16
_NBINS = _NB * _NB
_KMAX = 7


def _sc_body(x_hbm, y_hbm, sx_hbm, sy_hbm, w_hbm, out_hbm,
             xv, yv, sxv, syv, wv, acc):
    wid = lax.axis_index("s") * 2 + lax.axis_index("c")
    is_last = wid == _NW - 1
    base = wid * _CHUNK

    zeros16 = jnp.zeros((16,), jnp.float32)
    for buf in (xv, yv, sxv, syv, wv):
        buf[pl.ds(3120, 16)] = zeros16

    @pl.when(is_last)
    def _():
        for buf in (xv, yv, sxv, syv, wv):
            for r in range(3024, 3136, 16):
                buf[pl.ds(r, 16)] = zeros16

    @pl.when(jnp.logical_not(is_last))
    def _():
        pltpu.sync_copy(x_hbm.at[pl.ds(base, _CHUNK)], xv.at[pl.ds(0, _CHUNK)])
        pltpu.sync_copy(y_hbm.at[pl.ds(base, _CHUNK)], yv.at[pl.ds(0, _CHUNK)])
        pltpu.sync_copy(sx_hbm.at[pl.ds(base, _CHUNK)], sxv.at[pl.ds(0, _CHUNK)])
        pltpu.sync_copy(sy_hbm.at[pl.ds(base, _CHUNK)], syv.at[pl.ds(0, _CHUNK)])
        pltpu.sync_copy(w_hbm.at[pl.ds(base, _CHUNK)], wv.at[pl.ds(0, _CHUNK)])

    @pl.when(is_last)
    def _():
        pltpu.sync_copy(x_hbm.at[pl.ds(base, _LAST)], xv.at[pl.ds(0, _LAST)])
        pltpu.sync_copy(y_hbm.at[pl.ds(base, _LAST)], yv.at[pl.ds(0, _LAST)])
        pltpu.sync_copy(sx_hbm.at[pl.ds(base, _LAST)], sxv.at[pl.ds(0, _LAST)])
        pltpu.sync_copy(sy_hbm.at[pl.ds(base, _LAST)], syv.at[pl.ds(0, _LAST)])
        pltpu.sync_copy(w_hbm.at[pl.ds(base, _LAST)], wv.at[pl.ds(0, _LAST)])

    zero16 = jnp.zeros((16,), jnp.float32)

    @plsc.parallel_loop(0, 512, unroll=4)
    def zero_row(i):
        for k in range(8):
            acc[i, pl.ds(k * 16, 16)] = zero16

    @plsc.parallel_loop(0, _GROUPS + 1, unroll=2)
    def group_body(g):
        s = g * 16
        x = xv[pl.ds(s, 16)]
        y = yv[pl.ds(s, 16)]
        sx = jnp.maximum(sxv[pl.ds(s, 16)], _MIN_SIZE)
        sy = jnp.maximum(syv[pl.ds(s, 16)], _MIN_SIZE)
        w = wv[pl.ds(s, 16)]
        hx = 0.5 * sx
        hy = 0.5 * sy
        x_min = x - hx
        x_max = x + hx
        y_min = y - hy
        y_max = y + hy
        dens = (w * _SCALE) / (sx * sy)
        ix0 = (x_min * _INV_BS + 1024.0).astype(jnp.int32) - 1024
        iy0 = (y_min * _INV_BS + 1024.0).astype(jnp.int32) - 1024
        xlo0 = ix0.astype(jnp.float32) * _BS
        ylo0 = iy0.astype(jnp.float32) * _BS

        hi = []
        lo_col = []
        py = []
        my = []
        for dy in range(_KMAX):
            by = iy0 + dy
            lo = ylo0 + dy * _BS
            oy = jnp.maximum(
                jnp.minimum(y_max, lo + _BS) - jnp.maximum(y_min, lo), 0.0)
            py.append(oy)
            my.append(lax.bitcast_convert_type(by, jnp.uint32) < _NB)
            hi.append(by >> 7)
            lo_col.append(by & 127)

        for dx in range(_KMAX):
            bx = ix0 + dx
            lo = xlo0 + dx * _BS
            ox = jnp.maximum(
                jnp.minimum(x_max, lo + _BS) - jnp.maximum(x_min, lo), 0.0)
            pxd = dens * ox
            mxd = lax.bitcast_convert_type(bx, jnp.uint32) < _NB
            row2 = bx * 2
            for dy in range(_KMAX):
                row = row2 + hi[dy]
                val = pxd * py[dy]
                m = mxd & my[dy]
                plsc.addupdate_scatter(acc, [row, lo_col[dy]], val, mask=m)

    pltpu.sync_copy(acc, out_hbm.at[wid])


@jax.jit
def _sc_maps(x, y, sx, sy, w):
    mesh = plsc.VectorSubcoreMesh(core_axis_name="c", subcore_axis_name="s")
    return pl.kernel(
        _sc_body,
        out_type=jax.ShapeDtypeStruct((_NW, 2 * _NB, _NB // 2), jnp.float32),
        mesh=mesh,
        compiler_params=pltpu.CompilerParams(needs_layout_passes=False),
        scratch_types=[
            pltpu.VMEM((_GROUPS * 16 + 16,), jnp.float32),
            pltpu.VMEM((_GROUPS * 16 + 16,), jnp.float32),
            pltpu.VMEM((_GROUPS * 16 + 16,), jnp.float32),
            pltpu.VMEM((_GROUPS * 16 + 16,), jnp.float32),
            pltpu.VMEM((_GROUPS * 16 + 16,), jnp.float32),
            pltpu.VMEM((2 * _NB, _NB // 2), jnp.float32),
        ],
    )(x, y, sx, sy, w)


def _reduce_body(maps_ref, out_ref):
    out_ref[...] = jnp.sum(maps_ref[...], axis=0).reshape(_NB, _NB)


@jax.jit
def _reduce(maps):
    return pl.pallas_call(
        _reduce_body,
        out_shape=jax.ShapeDtypeStruct((_NB, _NB), jnp.float32),
    )(maps)


def kernel(inst_sizes, inst_pos, inst_pin_weights):
    maps = _sc_maps(inst_pos[:, 0], inst_pos[:, 1],
                    inst_sizes[:, 0], inst_sizes[:, 1], inst_pin_weights)
    return _reduce(maps)


# speedup vs baseline: 1.7993x; 1.7993x over previous
import jax
import jax.numpy as jnp
from jax import lax
from jax.experimental import pallas as pl
from jax.experimental.pallas import tpu as pltpu
from jax.experimental.pallas import tpu_sc as plsc

_N = 100000
_NB = 256
_BS = 1.0 / _NB
_INV_BS = float(_NB)
_MIN_SIZE = _BS * 1.4142135
_SCALE = 1.0 / (_BS * _BS * 100.0)
_NW = 32
_CHUNK = 3128
_LAST = _N - (_NW - 1) * _CHUNK
_GROUPS = _CHUNK // 16
_NBINS = _NB * _NB
_KMAX = 7


def _sc_body(x_hbm, y_hbm, sx_hbm, sy_hbm, w_hbm, out_hbm,
             xv, yv, sxv, syv, wv, acc):
    wid = lax.axis_index("s") * 2 + lax.axis_index("c")
    is_last = wid == _NW - 1
    base = wid * _CHUNK

    zeros16 = jnp.zeros((16,), jnp.float32)
    for buf in (xv, yv, sxv, syv, wv):
        buf[pl.ds(3120, 16)] = zeros16

    @pl.when(is_last)
    def _():
        for buf in (xv, yv, sxv, syv, wv):
            for r in range(3024, 3136, 16):
                buf[pl.ds(r, 16)] = zeros16

    @pl.when(jnp.logical_not(is_last))
    def _():
        pltpu.sync_copy(x_hbm.at[pl.ds(base, _CHUNK)], xv.at[pl.ds(0, _CHUNK)])
        pltpu.sync_copy(y_hbm.at[pl.ds(base, _CHUNK)], yv.at[pl.ds(0, _CHUNK)])
        pltpu.sync_copy(sx_hbm.at[pl.ds(base, _CHUNK)], sxv.at[pl.ds(0, _CHUNK)])
        pltpu.sync_copy(sy_hbm.at[pl.ds(base, _CHUNK)], syv.at[pl.ds(0, _CHUNK)])
        pltpu.sync_copy(w_hbm.at[pl.ds(base, _CHUNK)], wv.at[pl.ds(0, _CHUNK)])

    @pl.when(is_last)
    def _():
        pltpu.sync_copy(x_hbm.at[pl.ds(base, _LAST)], xv.at[pl.ds(0, _LAST)])
        pltpu.sync_copy(y_hbm.at[pl.ds(base, _LAST)], yv.at[pl.ds(0, _LAST)])
        pltpu.sync_copy(sx_hbm.at[pl.ds(base, _LAST)], sxv.at[pl.ds(0, _LAST)])
        pltpu.sync_copy(sy_hbm.at[pl.ds(base, _LAST)], syv.at[pl.ds(0, _LAST)])
        pltpu.sync_copy(w_hbm.at[pl.ds(base, _LAST)], wv.at[pl.ds(0, _LAST)])

    zero16 = jnp.zeros((16,), jnp.float32)

    @plsc.parallel_loop(0, _NBINS // 16, unroll=4)
    def zero_body(i):
        acc[pl.ds(i * 16, 16)] = zero16

    @plsc.parallel_loop(0, _GROUPS + 1)
    def group_body(g):
        s = g * 16
        x = xv[pl.ds(s, 16)]
        y = yv[pl.ds(s, 16)]
        sx = jnp.maximum(sxv[pl.ds(s, 16)], _MIN_SIZE)
        sy = jnp.maximum(syv[pl.ds(s, 16)], _MIN_SIZE)
        w = wv[pl.ds(s, 16)]
        hx = 0.5 * sx
        hy = 0.5 * sy
        x_min = x - hx
        x_max = x + hx
        y_min = y - hy
        y_max = y + hy
        dens = (w * _SCALE) / (sx * sy)
        ix0 = (x_min * _INV_BS + 1024.0).astype(jnp.int32) - 1024
        iy0 = (y_min * _INV_BS + 1024.0).astype(jnp.int32) - 1024
        xlo0 = ix0.astype(jnp.float32) * _BS
        ylo0 = iy0.astype(jnp.float32) * _BS

        col = []
        py = []
        ylo = ylo0
        for dy in range(_KMAX):
            by = iy0 + dy
            yhi = ylo0 + (dy + 1) * _BS
            oy = jnp.maximum(
                jnp.minimum(y_max, yhi) - jnp.maximum(y_min, ylo), 0.0)
            ylo = yhi
            in_y = lax.bitcast_convert_type(by, jnp.uint32) < _NB
            py.append(jnp.where(in_y, oy, 0.0))
            col.append(jnp.clip(by, 0, _NB - 1))

        xlo = xlo0
        for dx in range(_KMAX):
            bx = ix0 + dx
            xhi = xlo0 + (dx + 1) * _BS
            ox = jnp.maximum(
                jnp.minimum(x_max, xhi) - jnp.maximum(x_min, xlo), 0.0)
            xlo = xhi
            in_x = lax.bitcast_convert_type(bx, jnp.uint32) < _NB
            pxd = jnp.where(in_x, dens * ox, 0.0)
            rowbase = jnp.clip(bx, 0, _NB - 1) * _NB
            for dy in range(_KMAX):
                idx = rowbase + col[dy]
                val = pxd * py[dy]
                plsc.addupdate_scatter(acc, [idx], val)

    pltpu.sync_copy(acc, out_hbm.at[pl.ds(wid * _NBINS, _NBINS)])


@jax.jit
def _sc_maps(x, y, sx, sy, w):
    mesh = plsc.VectorSubcoreMesh(core_axis_name="c", subcore_axis_name="s")
    return pl.kernel(
        _sc_body,
        out_type=jax.ShapeDtypeStruct((_NW * _NBINS,), jnp.float32),
        mesh=mesh,
        compiler_params=pltpu.CompilerParams(needs_layout_passes=False),
        scratch_types=[
            pltpu.VMEM((3136,), jnp.float32),
            pltpu.VMEM((3136,), jnp.float32),
            pltpu.VMEM((3136,), jnp.float32),
            pltpu.VMEM((3136,), jnp.float32),
            pltpu.VMEM((3136,), jnp.float32),
            pltpu.VMEM((_NBINS,), jnp.float32),
        ],
    )(x, y, sx, sy, w)


def _reduce_body(maps_ref, out_ref):
    out_ref[...] = jnp.sum(maps_ref[...], axis=0).reshape(_NB, _NB)


@jax.jit
def _reduce(maps):
    return pl.pallas_call(
        _reduce_body,
        out_shape=jax.ShapeDtypeStruct((_NB, _NB), jnp.float32),
    )(maps.reshape(_NW, 2 * _NB, _NB // 2))


def kernel(inst_sizes, inst_pos, inst_pin_weights):
    maps = _sc_maps(inst_pos[:, 0], inst_pos[:, 1],
                    inst_sizes[:, 0], inst_sizes[:, 1], inst_pin_weights)
    return _reduce(maps)


# speedup vs baseline: 1.9263x; 1.0706x over previous
import jax
import jax.numpy as jnp
from jax import lax
from jax.experimental import pallas as pl
from jax.experimental.pallas import tpu as pltpu
from jax.experimental.pallas import tpu_sc as plsc

_N = 100000
_NB = 256
_BS = 1.0 / _NB
_INV_BS = float(_NB)
_MIN_SIZE = _BS * 1.4142135
_SCALE = 1.0 / (_BS * _BS * 100.0)
_NW = 32
_CHUNK = 3128
_LAST = _N - (_NW - 1) * _CHUNK
_GROUPS = _CHUNK // 16
_NBINS = _NB * _NB
_KMAX = 7


def _sc_body(x_hbm, y_hbm, sx_hbm, sy_hbm, w_hbm, out_hbm,
             xv, yv, sxv, syv, wv, rix, acc):
    wid = lax.axis_index("s") * 2 + lax.axis_index("c")
    is_last = wid == _NW - 1
    base = wid * _CHUNK

    zeros16 = jnp.zeros((16,), jnp.float32)
    for buf in (xv, yv, sxv, syv, wv):
        buf[pl.ds(3120, 16)] = zeros16

    @pl.when(is_last)
    def _():
        for buf in (xv, yv, sxv, syv, wv):
            for r in range(3024, 3136, 16):
                buf[pl.ds(r, 16)] = zeros16

    @pl.when(jnp.logical_not(is_last))
    def _():
        pltpu.sync_copy(x_hbm.at[pl.ds(base, _CHUNK)], xv.at[pl.ds(0, _CHUNK)])
        pltpu.sync_copy(y_hbm.at[pl.ds(base, _CHUNK)], yv.at[pl.ds(0, _CHUNK)])
        pltpu.sync_copy(sx_hbm.at[pl.ds(base, _CHUNK)], sxv.at[pl.ds(0, _CHUNK)])
        pltpu.sync_copy(sy_hbm.at[pl.ds(base, _CHUNK)], syv.at[pl.ds(0, _CHUNK)])
        pltpu.sync_copy(w_hbm.at[pl.ds(base, _CHUNK)], wv.at[pl.ds(0, _CHUNK)])

    @pl.when(is_last)
    def _():
        pltpu.sync_copy(x_hbm.at[pl.ds(base, _LAST)], xv.at[pl.ds(0, _LAST)])
        pltpu.sync_copy(y_hbm.at[pl.ds(base, _LAST)], yv.at[pl.ds(0, _LAST)])
        pltpu.sync_copy(sx_hbm.at[pl.ds(base, _LAST)], sxv.at[pl.ds(0, _LAST)])
        pltpu.sync_copy(sy_hbm.at[pl.ds(base, _LAST)], syv.at[pl.ds(0, _LAST)])
        pltpu.sync_copy(w_hbm.at[pl.ds(base, _LAST)], wv.at[pl.ds(0, _LAST)])

    zero16 = jnp.zeros((16,), jnp.float32)
    iota16 = lax.iota(jnp.int32, 16)
    pad16 = jnp.full((16,), 3120, jnp.int32)

    @plsc.parallel_loop(0, _NBINS // 16, unroll=4)
    def zero_body(i):
        acc[pl.ds(i * 16, 16)] = zero16

    @plsc.parallel_loop(0, (_GROUPS + 2) * 16 // 16)
    def prefill_body(i):
        rix[pl.ds(i * 16, 16)] = pad16

    def axes_setup(x, y, sx_r, sy_r, w):
        sx = jnp.maximum(sx_r, _MIN_SIZE)
        sy = jnp.maximum(sy_r, _MIN_SIZE)
        hx = 0.5 * sx
        hy = 0.5 * sy
        x_min = x - hx
        x_max = x + hx
        y_min = y - hy
        y_max = y + hy
        dens = (w * _SCALE) / (sx * sy)
        ix0 = (x_min * _INV_BS + 1024.0).astype(jnp.int32) - 1024
        iy0 = (y_min * _INV_BS + 1024.0).astype(jnp.int32) - 1024
        return x_min, x_max, y_min, y_max, dens, ix0, iy0

    def y_side(y_min, y_max, iy0, dys):
        ylo0 = iy0.astype(jnp.float32) * _BS
        col = {}
        py = {}
        for dy in dys:
            by = iy0 + dy
            oy = jnp.maximum(
                jnp.minimum(y_max, ylo0 + (dy + 1) * _BS)
                - jnp.maximum(y_min, ylo0 + dy * _BS), 0.0)
            in_y = lax.bitcast_convert_type(by, jnp.uint32) < _NB
            py[dy] = jnp.where(in_y, oy, 0.0)
            col[dy] = jnp.clip(by, 0, _NB - 1)
        return col, py

    def x_scatter(x_min, x_max, ix0, dens, col, py, dxs, pairs):
        xlo0 = ix0.astype(jnp.float32) * _BS
        for dx in dxs:
            bx = ix0 + dx
            ox = jnp.maximum(
                jnp.minimum(x_max, xlo0 + (dx + 1) * _BS)
                - jnp.maximum(x_min, xlo0 + dx * _BS), 0.0)
            in_x = lax.bitcast_convert_type(bx, jnp.uint32) < _NB
            pxd = jnp.where(in_x, dens * ox, 0.0)
            rowbase = jnp.clip(bx, 0, _NB - 1) * _NB
            for dy in pairs(dx):
                idx = rowbase + col[dy]
                val = pxd * py[dy]
                plsc.addupdate_scatter(acc, [idx], val)

    @plsc.parallel_loop(0, _GROUPS + 1, carry=jnp.int32(0))
    def group_body(g, n_res):
        s = g * 16
        x = xv[pl.ds(s, 16)]
        y = yv[pl.ds(s, 16)]
        sx_r = sxv[pl.ds(s, 16)]
        sy_r = syv[pl.ds(s, 16)]
        w = wv[pl.ds(s, 16)]
        x_min, x_max, y_min, y_max, dens, ix0, iy0 = axes_setup(
            x, y, sx_r, sy_r, w)
        ix1 = (x_max * _INV_BS + 1024.0).astype(jnp.int32) - 1024
        iy1 = (y_max * _INV_BS + 1024.0).astype(jnp.int32) - 1024
        big = ((ix1 - ix0) >= _KMAX - 1) | ((iy1 - iy0) >= _KMAX - 1)
        plsc.store_compressed(rix.at[pl.ds(n_res, 16)], s + iota16, mask=big)
        cnt = plsc.all_reduce_population_count(big)

        col, py = y_side(y_min, y_max, iy0, range(_KMAX - 1))
        x_scatter(x_min, x_max, ix0, dens, col, py,
                  range(_KMAX - 1), lambda dx: range(_KMAX - 1))
        return n_res + cnt[0]

    n_res = group_body
    ng2 = (n_res + 15) >> 4

    def res_body(g, c):
        idxv = rix[pl.ds(g * 16, 16)]
        x = plsc.load_gather(xv, [idxv])
        y = plsc.load_gather(yv, [idxv])
        sx_r = plsc.load_gather(sxv, [idxv])
        sy_r = plsc.load_gather(syv, [idxv])
        w = plsc.load_gather(wv, [idxv])
        x_min, x_max, y_min, y_max, dens, ix0, iy0 = axes_setup(
            x, y, sx_r, sy_r, w)
        col, py = y_side(y_min, y_max, iy0, range(_KMAX))
        x_scatter(x_min, x_max, ix0, dens, col, py, range(_KMAX),
                  lambda dx: range(_KMAX) if dx == _KMAX - 1
                  else [_KMAX - 1])
        return c

    lax.fori_loop(0, ng2, res_body, 0)

    pltpu.sync_copy(acc, out_hbm.at[pl.ds(wid * _NBINS, _NBINS)])


@jax.jit
def _sc_maps(x, y, sx, sy, w):
    mesh = plsc.VectorSubcoreMesh(core_axis_name="c", subcore_axis_name="s")
    return pl.kernel(
        _sc_body,
        out_type=jax.ShapeDtypeStruct((_NW * _NBINS,), jnp.float32),
        mesh=mesh,
        compiler_params=pltpu.CompilerParams(needs_layout_passes=False),
        scratch_types=[
            pltpu.VMEM((3136,), jnp.float32),
            pltpu.VMEM((3136,), jnp.float32),
            pltpu.VMEM((3136,), jnp.float32),
            pltpu.VMEM((3136,), jnp.float32),
            pltpu.VMEM((3136,), jnp.float32),
            pltpu.VMEM(((_GROUPS + 2) * 16,), jnp.int32),
            pltpu.VMEM((_NBINS,), jnp.float32),
        ],
    )(x, y, sx, sy, w)


def _reduce_body(maps_ref, out_ref):
    out_ref[...] = jnp.sum(maps_ref[...], axis=0).reshape(_NB, _NB)


@jax.jit
def _reduce(maps):
    return pl.pallas_call(
        _reduce_body,
        out_shape=jax.ShapeDtypeStruct((_NB, _NB), jnp.float32),
    )(maps.reshape(_NW, 2 * _NB, _NB // 2))


def kernel(inst_sizes, inst_pos, inst_pin_weights):
    maps = _sc_maps(inst_pos[:, 0], inst_pos[:, 1],
                    inst_sizes[:, 0], inst_sizes[:, 1], inst_pin_weights)
    return _reduce(maps)
